# baseline (speedup 1.0000x reference)
import functools

import jax
import jax.numpy as jnp
from jax import lax
from jax.experimental import pallas as pl
from jax.experimental.pallas import tpu as pltpu, tpu_sc as plsc

B, N, E, DIN, DOUT = 4, 10000, 320000, 128, 128
NTILES = 16
NCORES = 2
CHUNK = 128
C = 168
EPAD = NTILES * C * CHUNK
FG = DIN // 16
ZROWS = 624
ZSTEP = 104


def _sc_spmm(xflat, epk, wp):
    mesh = plsc.VectorSubcoreMesh(core_axis_name="c", subcore_axis_name="s")

    @functools.partial(
        pl.kernel,
        out_type=jax.ShapeDtypeStruct((B, N, DIN), jnp.float32),
        mesh=mesh,
        compiler_params=pltpu.CompilerParams(needs_layout_passes=False),
        scratch_types=[
            pltpu.MemorySpace.VMEM_SHARED((N, DIN), jnp.float32),
            pltpu.MemorySpace.VMEM((4, 2, CHUNK), jnp.int32),
            pltpu.MemorySpace.VMEM((3 * CHUNK,), jnp.float32),
            pltpu.MemorySpace.VMEM((3, CHUNK), jnp.int32),
            pltpu.MemorySpace.VMEM((3, CHUNK, DIN), jnp.float32),
            [pltpu.SemaphoreType.DMA] * 3,
            [pltpu.SemaphoreType.DMA] * 4,
            [pltpu.SemaphoreType.DMA] * 3,
        ],
    )
    def k(x_hbm, e_hbm, w_hbm, out_hbm, acc, ering, wring, sidx, rows,
          gsems, esems, ssems):
        s = lax.axis_index("s")
        cid = lax.axis_index("c")

        zero16 = jnp.zeros((16,), jnp.float32)

        def edma(slot, wslot, c):
            a = pltpu.make_async_copy(
                e_hbm.at[s, c], ering.at[slot], esems[slot])
            b_ = pltpu.make_async_copy(
                w_hbm.at[s, c],
                wring.at[pl.ds(wslot * CHUNK, CHUNK)], esems[slot])
            return a, b_

        def estart(slot, wslot, c):
            a, b_ = edma(slot, wslot, c)
            a.start()
            b_.start()

        def ewait(slot, wslot, c):
            a, b_ = edma(slot, wslot, c)
            a.wait()
            b_.wait()

        def build_sidx(slot, bi, bnv):
            for f in range(CHUNK // 16):
                sl = pl.ds(f * 16, 16)
                sidx[bi, sl] = ering[slot, 0, sl] + bnv

        def gstart(bi):
            pltpu.make_async_copy(
                x_hbm.at[sidx.at[bi]], rows.at[bi], gsems[bi]).start()

        def gwait(bi):
            pltpu.make_async_copy(
                x_hbm.at[sidx.at[bi]], rows.at[bi], gsems[bi]).wait()

        def scale(bi):
            def sbody(i, ivec):
                w16 = plsc.load_gather(wring, [ivec])
                for f in range(FG):
                    sl = pl.ds(f * 16, 16)
                    rows[bi, i, sl] = rows[bi, i, sl] * w16
                return ivec + 1

            lax.fori_loop(0, CHUNK, sbody,
                          jnp.full((16,), bi * CHUNK, jnp.int32), unroll=2)

        def sstart(bi, slot):
            pltpu.async_copy(rows.at[bi], acc.at[ering.at[slot, 1]],
                             ssems[bi], add=True)

        def swait(bi):
            pltpu.make_async_copy(rows.at[bi], acc.at[ering.at[0, 1]],
                                  ssems[bi]).wait()

        zbase = s * ZROWS
        is_last_tile = s == NTILES - 1
        NB = C // 12
        clive = (E // CHUNK + NTILES - 1 - s) // NTILES

        for p in range(B // NCORES):
            batch = p * NCORES + cid
            oidx = batch

            def zbody(r, _):
                for f in range(FG):
                    rows[0, r, pl.ds(f * 16, 16)] = zero16
                return 0
            lax.fori_loop(0, ZSTEP, zbody, 0)
            for z in range(ZROWS // ZSTEP):
                pltpu.sync_copy(rows.at[0].at[pl.ds(0, ZSTEP)],
                                acc.at[pl.ds(zbase + z * ZSTEP, ZSTEP)])

            @pl.when(is_last_tile)
            def _():
                pltpu.sync_copy(rows.at[0].at[pl.ds(0, 16)],
                                acc.at[pl.ds(zbase + ZROWS, 16)])

            plsc.subcore_barrier()

            bnv = jnp.full((16,), batch * N, jnp.int32)

            for slot in range(3):
                estart(slot, slot, slot)
            for c in range(2):
                ewait(c, c, c)
                build_sidx(c, c, bnv)
                gstart(c)

            def body(kk, _):
                c0 = kk * 12
                last = kk == NB - 1
                for j in range(12):
                    bj = j % 3
                    b2 = (j + 2) % 3
                    slot = j % 4
                    nslot = (j + 2) % 4
                    pslot = (j - 1) % 4

                    pl.when(c0 + j < clive)(lambda bb=bj: gwait(bb))

                    @pl.when(c0 + j < clive)
                    def _(bb=bj, ss=slot):
                        scale(bb)
                        sstart(bb, ss)

                    def prep(nn=nslot, ww=(j + 2) % 3, cc=c0 + j + 2, rr=b2):
                        ewait(nn, ww, cc)
                        build_sidx(nn, rr, bnv)

                    def refill(ps=pslot, ww=j % 3, cc=c0 + j + 3):
                        estart(ps, ww, cc)

                    def launch(rr=b2, cc=c0 + j + 2):
                        pl.when(cc < clive)(lambda: gstart(rr))

                    prev_live = jnp.logical_and(c0 + j >= 1,
                                                c0 + j - 1 < clive)
                    if j == 0:
                        prep()
                        pl.when(prev_live)(lambda rr=b2: swait(rr))
                        refill()
                        launch()
                    elif j <= 8:
                        prep()
                        pl.when(prev_live)(lambda rr=b2: swait(rr))
                        refill()
                        launch()
                    elif j == 9:
                        prep()
                        pl.when(prev_live)(lambda rr=b2: swait(rr))
                        pl.when(jnp.logical_not(last))(refill)
                        launch()
                    else:
                        pl.when(prev_live)(lambda rr=b2: swait(rr))

                        def tail(pp=prep, rf=refill, ln=launch):
                            pp()
                            rf()
                            ln()
                        pl.when(jnp.logical_not(last))(tail)
                return 0

            lax.fori_loop(0, NB, body, 0)

            plsc.subcore_barrier()
            for z in range(ZROWS // ZSTEP):
                r0 = zbase + z * ZSTEP
                pltpu.sync_copy(acc.at[pl.ds(r0, ZSTEP)],
                                out_hbm.at[oidx, pl.ds(r0, ZSTEP)])

            @pl.when(is_last_tile)
            def _():
                pltpu.sync_copy(acc.at[pl.ds(zbase + ZROWS, 16)],
                                out_hbm.at[oidx, pl.ds(zbase + ZROWS, 16)])

            plsc.subcore_barrier()

    return k(xflat, epk, wp)


def _tc_transform(agg, W, b2d):
    BLK = 1000
    grid = (B, N // BLK)

    def body(a_ref, w_ref, b_ref, o_ref):
        acc = jnp.dot(a_ref[0], w_ref[...],
                      preferred_element_type=jnp.float32)
        o_ref[0] = jnp.maximum(acc + b_ref[...], 0.0)

    return pl.pallas_call(
        body,
        grid=grid,
        in_specs=[
            pl.BlockSpec((1, BLK, DIN), lambda bb, i: (bb, i, 0)),
            pl.BlockSpec((DIN, DOUT), lambda bb, i: (0, 0)),
            pl.BlockSpec((1, DOUT), lambda bb, i: (0, 0)),
        ],
        out_specs=pl.BlockSpec((1, BLK, DOUT), lambda bb, i: (bb, i, 0)),
        out_shape=jax.ShapeDtypeStruct((B, N, DOUT), jnp.float32),
    )(agg, W, b2d)


def kernel(x, edge_index, edge_weight, W, b):
    dst = edge_index[0].astype(jnp.int32)
    src = edge_index[1].astype(jnp.int32)
    w = edge_weight.astype(jnp.float32)

    pad = EPAD - E
    fill = (jnp.arange(pad, dtype=jnp.int32) * 16) % N
    srcp = jnp.concatenate([src, fill]).reshape(C, NTILES, 1, CHUNK)
    dstp = jnp.concatenate([dst, fill]).reshape(C, NTILES, 1, CHUNK)
    epk = jnp.concatenate([srcp, dstp], axis=2).transpose(1, 0, 2, 3)
    wp = jnp.concatenate([w, jnp.zeros((pad,), jnp.float32)]).reshape(
        C, NTILES, CHUNK).transpose(1, 0, 2)

    xflat = x.reshape(B * N, DIN)
    agg = _sc_spmm(xflat, epk, wp)
    return _tc_transform(agg, W, b.reshape(1, DOUT))


# speedup vs baseline: 1.0083x; 1.0083x over previous
import functools

import jax
import jax.numpy as jnp
from jax import lax
from jax.experimental import pallas as pl
from jax.experimental.pallas import tpu as pltpu, tpu_sc as plsc

B, N, E, DIN, DOUT = 4, 10000, 320000, 128, 128
NTILES = 16
NCORES = 2
CHUNK = 128
C = 168
EPAD = NTILES * C * CHUNK
FG = DIN // 16
ZROWS = 624
ZSTEP = 104


def _sc_spmm(xflat, epk, wp):
    mesh = plsc.VectorSubcoreMesh(core_axis_name="c", subcore_axis_name="s")

    @functools.partial(
        pl.kernel,
        out_type=jax.ShapeDtypeStruct((B, N, DIN), jnp.float32),
        mesh=mesh,
        compiler_params=pltpu.CompilerParams(needs_layout_passes=False),
        scratch_types=[
            pltpu.MemorySpace.VMEM_SHARED((N, DIN), jnp.float32),
            pltpu.MemorySpace.VMEM((4, 2, CHUNK), jnp.int32),
            pltpu.MemorySpace.VMEM((3 * CHUNK,), jnp.float32),
            pltpu.MemorySpace.VMEM((3, CHUNK), jnp.int32),
            pltpu.MemorySpace.VMEM((3, CHUNK, DIN), jnp.float32),
            [pltpu.SemaphoreType.DMA] * 3,
            [pltpu.SemaphoreType.DMA] * 4,
            [pltpu.SemaphoreType.DMA] * 3,
        ],
    )
    def k(x_hbm, e_hbm, w_hbm, out_hbm, acc, ering, wring, sidx, rows,
          gsems, esems, ssems):
        s = lax.axis_index("s")
        cid = lax.axis_index("c")

        zero16 = jnp.zeros((16,), jnp.float32)

        def edma(slot, wslot, c):
            a = pltpu.make_async_copy(
                e_hbm.at[s, c], ering.at[slot], esems[slot])
            b_ = pltpu.make_async_copy(
                w_hbm.at[s, c],
                wring.at[pl.ds(wslot * CHUNK, CHUNK)], esems[slot])
            return a, b_

        def estart(slot, wslot, c):
            a, b_ = edma(slot, wslot, c)
            a.start()
            b_.start()

        def ewait(slot, wslot, c):
            a, b_ = edma(slot, wslot, c)
            a.wait()
            b_.wait()

        def build_sidx(slot, bi, bnv):
            for f in range(CHUNK // 16):
                sl = pl.ds(f * 16, 16)
                sidx[bi, sl] = ering[slot, 0, sl] + bnv

        def gstart(bi):
            pltpu.make_async_copy(
                x_hbm.at[sidx.at[bi]], rows.at[bi], gsems[bi]).start()

        def gwait(bi):
            pltpu.make_async_copy(
                x_hbm.at[sidx.at[bi]], rows.at[bi], gsems[bi]).wait()

        def scale(bi):
            def sbody(i, ivec):
                w16 = plsc.load_gather(wring, [ivec])
                for f in range(FG):
                    sl = pl.ds(f * 16, 16)
                    rows[bi, i, sl] = rows[bi, i, sl] * w16
                return ivec + 1

            lax.fori_loop(0, CHUNK, sbody,
                          jnp.full((16,), bi * CHUNK, jnp.int32), unroll=2)

        def sstart(bi, slot):
            pltpu.async_copy(rows.at[bi], acc.at[ering.at[slot, 1]],
                             ssems[bi], add=True, priority=1)

        def swait(bi):
            pltpu.make_async_copy(rows.at[bi], acc.at[ering.at[0, 1]],
                                  ssems[bi]).wait()

        zbase = s * ZROWS
        is_last_tile = s == NTILES - 1
        NB = C // 12
        clive = (E // CHUNK + NTILES - 1 - s) // NTILES

        for p in range(B // NCORES):
            batch = p * NCORES + cid
            oidx = batch

            def zbody(r, _):
                for f in range(FG):
                    rows[0, r, pl.ds(f * 16, 16)] = zero16
                return 0
            lax.fori_loop(0, ZSTEP, zbody, 0)
            for z in range(ZROWS // ZSTEP):
                pltpu.sync_copy(rows.at[0].at[pl.ds(0, ZSTEP)],
                                acc.at[pl.ds(zbase + z * ZSTEP, ZSTEP)])

            @pl.when(is_last_tile)
            def _():
                pltpu.sync_copy(rows.at[0].at[pl.ds(0, 16)],
                                acc.at[pl.ds(zbase + ZROWS, 16)])

            plsc.subcore_barrier()

            bnv = jnp.full((16,), batch * N, jnp.int32)

            for slot in range(3):
                estart(slot, slot, slot)
            for c in range(2):
                ewait(c, c, c)
                build_sidx(c, c, bnv)
                gstart(c)

            def body(kk, _):
                c0 = kk * 12
                last = kk == NB - 1
                for j in range(12):
                    bj = j % 3
                    b2 = (j + 2) % 3
                    slot = j % 4
                    nslot = (j + 2) % 4
                    pslot = (j - 1) % 4

                    pl.when(c0 + j < clive)(lambda bb=bj: gwait(bb))

                    @pl.when(c0 + j < clive)
                    def _(bb=bj, ss=slot):
                        scale(bb)
                        sstart(bb, ss)

                    def prep(nn=nslot, ww=(j + 2) % 3, cc=c0 + j + 2, rr=b2):
                        ewait(nn, ww, cc)
                        build_sidx(nn, rr, bnv)

                    def refill(ps=pslot, ww=j % 3, cc=c0 + j + 3):
                        estart(ps, ww, cc)

                    def launch(rr=b2, cc=c0 + j + 2):
                        pl.when(cc < clive)(lambda: gstart(rr))

                    prev_live = jnp.logical_and(c0 + j >= 1,
                                                c0 + j - 1 < clive)
                    if j == 0:
                        prep()
                        pl.when(prev_live)(lambda rr=b2: swait(rr))
                        refill()
                        launch()
                    elif j <= 8:
                        prep()
                        pl.when(prev_live)(lambda rr=b2: swait(rr))
                        refill()
                        launch()
                    elif j == 9:
                        prep()
                        pl.when(prev_live)(lambda rr=b2: swait(rr))
                        pl.when(jnp.logical_not(last))(refill)
                        launch()
                    else:
                        pl.when(prev_live)(lambda rr=b2: swait(rr))

                        def tail(pp=prep, rf=refill, ln=launch):
                            pp()
                            rf()
                            ln()
                        pl.when(jnp.logical_not(last))(tail)
                return 0

            lax.fori_loop(0, NB, body, 0)

            plsc.subcore_barrier()
            for z in range(ZROWS // ZSTEP):
                r0 = zbase + z * ZSTEP
                pltpu.sync_copy(acc.at[pl.ds(r0, ZSTEP)],
                                out_hbm.at[oidx, pl.ds(r0, ZSTEP)])

            @pl.when(is_last_tile)
            def _():
                pltpu.sync_copy(acc.at[pl.ds(zbase + ZROWS, 16)],
                                out_hbm.at[oidx, pl.ds(zbase + ZROWS, 16)])

            plsc.subcore_barrier()

    return k(xflat, epk, wp)


def _tc_transform(agg, W, b2d):
    BLK = 1000
    grid = (B, N // BLK)

    def body(a_ref, w_ref, b_ref, o_ref):
        acc = jnp.dot(a_ref[0], w_ref[...],
                      preferred_element_type=jnp.float32)
        o_ref[0] = jnp.maximum(acc + b_ref[...], 0.0)

    return pl.pallas_call(
        body,
        grid=grid,
        in_specs=[
            pl.BlockSpec((1, BLK, DIN), lambda bb, i: (bb, i, 0)),
            pl.BlockSpec((DIN, DOUT), lambda bb, i: (0, 0)),
            pl.BlockSpec((1, DOUT), lambda bb, i: (0, 0)),
        ],
        out_specs=pl.BlockSpec((1, BLK, DOUT), lambda bb, i: (bb, i, 0)),
        out_shape=jax.ShapeDtypeStruct((B, N, DOUT), jnp.float32),
    )(agg, W, b2d)


def kernel(x, edge_index, edge_weight, W, b):
    dst = edge_index[0].astype(jnp.int32)
    src = edge_index[1].astype(jnp.int32)
    w = edge_weight.astype(jnp.float32)

    pad = EPAD - E
    fill = (jnp.arange(pad, dtype=jnp.int32) * 16) % N
    srcp = jnp.concatenate([src, fill]).reshape(C, NTILES, 1, CHUNK)
    dstp = jnp.concatenate([dst, fill]).reshape(C, NTILES, 1, CHUNK)
    epk = jnp.concatenate([srcp, dstp], axis=2).transpose(1, 0, 2, 3)
    wp = jnp.concatenate([w, jnp.zeros((pad,), jnp.float32)]).reshape(
        C, NTILES, CHUNK).transpose(1, 0, 2)

    xflat = x.reshape(B * N, DIN)
    agg = _sc_spmm(xflat, epk, wp)
    return _tc_transform(agg, W, b.reshape(1, DOUT))


# speedup vs baseline: 1.1023x; 1.0933x over previous
import functools

import jax
import jax.numpy as jnp
from jax import lax
from jax.experimental import pallas as pl
from jax.experimental.pallas import tpu as pltpu, tpu_sc as plsc

B, N, E, DIN, DOUT = 4, 10000, 320000, 128, 128
NTILES = 16
NCORES = 2
CHUNK = 128
C = 168
EPAD = NTILES * C * CHUNK
FG = DIN // 16
ZROWS = 624
ZSTEP = 104


def _sc_spmm(xflat, epk, wp):
    mesh = plsc.VectorSubcoreMesh(core_axis_name="c", subcore_axis_name="s")

    @functools.partial(
        pl.kernel,
        out_type=jax.ShapeDtypeStruct((B, N, DIN), jnp.float32),
        mesh=mesh,
        compiler_params=pltpu.CompilerParams(needs_layout_passes=False),
        scratch_types=[
            pltpu.MemorySpace.VMEM_SHARED((N, DIN), jnp.float32),
            pltpu.MemorySpace.VMEM((4, 2, CHUNK), jnp.int32),
            pltpu.MemorySpace.VMEM((3 * CHUNK,), jnp.float32),
            pltpu.MemorySpace.VMEM((3, CHUNK), jnp.int32),
            pltpu.MemorySpace.VMEM((3, CHUNK, DIN), jnp.float32),
            [pltpu.SemaphoreType.DMA] * 3,
            [pltpu.SemaphoreType.DMA] * 4,
            [pltpu.SemaphoreType.DMA] * 3,
        ],
    )
    def k(x_hbm, e_hbm, w_hbm, out_hbm, acc, ering, wring, sidx, rows,
          gsems, esems, ssems):
        s = lax.axis_index("s")
        cid = lax.axis_index("c")

        zero16 = jnp.zeros((16,), jnp.float32)

        def edma(slot, wslot, c):
            a = pltpu.make_async_copy(
                e_hbm.at[s, c], ering.at[slot], esems[slot])
            b_ = pltpu.make_async_copy(
                w_hbm.at[s, c],
                wring.at[pl.ds(wslot * CHUNK, CHUNK)], esems[slot])
            return a, b_

        def estart(slot, wslot, c):
            a, b_ = edma(slot, wslot, c)
            a.start()
            b_.start()

        def ewait(slot, wslot, c):
            a, b_ = edma(slot, wslot, c)
            a.wait()
            b_.wait()

        def build_sidx(slot, bi, bnv):
            for f in range(CHUNK // 16):
                sl = pl.ds(f * 16, 16)
                sidx[bi, sl] = ering[slot, 0, sl] + bnv

        def gstart(bi):
            pltpu.make_async_copy(
                x_hbm.at[sidx.at[bi]], rows.at[bi], gsems[bi]).start()

        def gwait(bi):
            pltpu.make_async_copy(
                x_hbm.at[sidx.at[bi]], rows.at[bi], gsems[bi]).wait()

        def scale(bi):
            def sbody(i, ivec):
                w16 = plsc.load_gather(wring, [ivec])
                for f in range(FG):
                    sl = pl.ds(f * 16, 16)
                    rows[bi, i, sl] = rows[bi, i, sl] * w16
                return ivec + 1

            plsc.parallel_loop(0, CHUNK, 1, unroll=2,
                               carry=jnp.full((16,), bi * CHUNK,
                                              jnp.int32))(sbody)

        def sstart(bi, slot):
            pltpu.async_copy(rows.at[bi], acc.at[ering.at[slot, 1]],
                             ssems[bi], add=True, priority=1)

        def swait(bi):
            pltpu.make_async_copy(rows.at[bi], acc.at[ering.at[0, 1]],
                                  ssems[bi]).wait()

        zbase = s * ZROWS
        is_last_tile = s == NTILES - 1
        NB = C // 12
        clive = (E // CHUNK + NTILES - 1 - s) // NTILES

        for p in range(B // NCORES):
            batch = p * NCORES + cid
            oidx = batch

            def zbody(r, _):
                for f in range(FG):
                    rows[0, r, pl.ds(f * 16, 16)] = zero16
                return 0
            lax.fori_loop(0, ZSTEP, zbody, 0)
            for z in range(ZROWS // ZSTEP):
                pltpu.sync_copy(rows.at[0].at[pl.ds(0, ZSTEP)],
                                acc.at[pl.ds(zbase + z * ZSTEP, ZSTEP)])

            @pl.when(is_last_tile)
            def _():
                pltpu.sync_copy(rows.at[0].at[pl.ds(0, 16)],
                                acc.at[pl.ds(zbase + ZROWS, 16)])

            plsc.subcore_barrier()

            bnv = jnp.full((16,), batch * N, jnp.int32)

            for slot in range(3):
                estart(slot, slot, slot)
            for c in range(2):
                ewait(c, c, c)
                build_sidx(c, c, bnv)
                gstart(c)

            def body(kk, _):
                c0 = kk * 12
                last = kk == NB - 1
                for j in range(12):
                    bj = j % 3
                    b2 = (j + 2) % 3
                    slot = j % 4
                    nslot = (j + 2) % 4
                    pslot = (j - 1) % 4

                    pl.when(c0 + j < clive)(lambda bb=bj: gwait(bb))

                    @pl.when(c0 + j < clive)
                    def _(bb=bj, ss=slot):
                        scale(bb)
                        sstart(bb, ss)

                    def prep(nn=nslot, ww=(j + 2) % 3, cc=c0 + j + 2, rr=b2):
                        ewait(nn, ww, cc)
                        build_sidx(nn, rr, bnv)

                    def refill(ps=pslot, ww=j % 3, cc=c0 + j + 3):
                        estart(ps, ww, cc)

                    def launch(rr=b2, cc=c0 + j + 2):
                        pl.when(cc < clive)(lambda: gstart(rr))

                    prev_live = jnp.logical_and(c0 + j >= 1,
                                                c0 + j - 1 < clive)
                    if j == 0:
                        prep()
                        pl.when(prev_live)(lambda rr=b2: swait(rr))
                        refill()
                        launch()
                    elif j <= 8:
                        prep()
                        pl.when(prev_live)(lambda rr=b2: swait(rr))
                        refill()
                        launch()
                    elif j == 9:
                        prep()
                        pl.when(prev_live)(lambda rr=b2: swait(rr))
                        pl.when(jnp.logical_not(last))(refill)
                        launch()
                    else:
                        pl.when(prev_live)(lambda rr=b2: swait(rr))

                        def tail(pp=prep, rf=refill, ln=launch):
                            pp()
                            rf()
                            ln()
                        pl.when(jnp.logical_not(last))(tail)
                return 0

            lax.fori_loop(0, NB, body, 0)

            plsc.subcore_barrier()
            for z in range(ZROWS // ZSTEP):
                r0 = zbase + z * ZSTEP
                pltpu.sync_copy(acc.at[pl.ds(r0, ZSTEP)],
                                out_hbm.at[oidx, pl.ds(r0, ZSTEP)])

            @pl.when(is_last_tile)
            def _():
                pltpu.sync_copy(acc.at[pl.ds(zbase + ZROWS, 16)],
                                out_hbm.at[oidx, pl.ds(zbase + ZROWS, 16)])

            plsc.subcore_barrier()

    return k(xflat, epk, wp)


def _tc_transform(agg, W, b2d):
    BLK = 1000
    grid = (B, N // BLK)

    def body(a_ref, w_ref, b_ref, o_ref):
        acc = jnp.dot(a_ref[0], w_ref[...],
                      preferred_element_type=jnp.float32)
        o_ref[0] = jnp.maximum(acc + b_ref[...], 0.0)

    return pl.pallas_call(
        body,
        grid=grid,
        in_specs=[
            pl.BlockSpec((1, BLK, DIN), lambda bb, i: (bb, i, 0)),
            pl.BlockSpec((DIN, DOUT), lambda bb, i: (0, 0)),
            pl.BlockSpec((1, DOUT), lambda bb, i: (0, 0)),
        ],
        out_specs=pl.BlockSpec((1, BLK, DOUT), lambda bb, i: (bb, i, 0)),
        out_shape=jax.ShapeDtypeStruct((B, N, DOUT), jnp.float32),
    )(agg, W, b2d)


def kernel(x, edge_index, edge_weight, W, b):
    dst = edge_index[0].astype(jnp.int32)
    src = edge_index[1].astype(jnp.int32)
    w = edge_weight.astype(jnp.float32)

    pad = EPAD - E
    fill = (jnp.arange(pad, dtype=jnp.int32) * 16) % N
    srcp = jnp.concatenate([src, fill]).reshape(C, NTILES, 1, CHUNK)
    dstp = jnp.concatenate([dst, fill]).reshape(C, NTILES, 1, CHUNK)
    epk = jnp.concatenate([srcp, dstp], axis=2).transpose(1, 0, 2, 3)
    wp = jnp.concatenate([w, jnp.zeros((pad,), jnp.float32)]).reshape(
        C, NTILES, CHUNK).transpose(1, 0, 2)

    xflat = x.reshape(B * N, DIN)
    agg = _sc_spmm(xflat, epk, wp)
    return _tc_transform(agg, W, b.reshape(1, DOUT))
